# baseline (speedup 1.0000x reference)
import functools

import jax
import jax.numpy as jnp
from jax import lax
from jax.experimental import pallas as pl
from jax.experimental.pallas import tpu as pltpu
from jax.experimental.pallas import tpu_sc as plsc

NUM_CORES = 2
NUM_SUBCORES = 16
NUM_WORKERS = NUM_CORES * NUM_SUBCORES


@functools.cache
def _make_kernel(batch, seq_len, vocab, embed):
  assert batch % (128 * NUM_WORKERS) == 0 or batch == 128 * NUM_WORKERS
  assert embed == 32
  mesh = plsc.VectorSubcoreMesh(
      core_axis_name="c", subcore_axis_name="s",
      num_cores=NUM_CORES, num_subcores=NUM_SUBCORES)

  @functools.partial(
      pl.kernel,
      out_type=jax.ShapeDtypeStruct(
          (seq_len, embed // 8, batch // 128, 8, 128), jnp.float32),
      mesh=mesh,
      compiler_params=pltpu.CompilerParams(
          use_tc_tiling_on_sc=False, needs_layout_passes=False),
      scratch_types=[
          pltpu.VMEM((128,), jnp.int32),
          pltpu.VMEM((128, embed), jnp.float32),
          pltpu.VMEM((embed // 8, 8, 128), jnp.float32),
          pltpu.VMEM((seq_len, embed), jnp.float32),
          pltpu.SemaphoreType.DMA,
      ],
  )
  def k(table_hbm, idx_hbm, pos_hbm, out_hbm, idx_v, g_v, t_v, pos_v, sem):
    wid = lax.axis_index("s") * NUM_CORES + lax.axis_index("c")
    pltpu.sync_copy(pos_hbm, pos_v)
    iota = lax.iota(jnp.int32, 16)

    def body(l, _):
      off = pl.multiple_of(l * batch + wid * 128, 128)
      pltpu.sync_copy(idx_hbm.at[pl.ds(off, 128)], idx_v)
      pltpu.async_copy(table_hbm.at[idx_v], g_v, sem).wait()
      bl = jnp.broadcast_to(l, (16,)).astype(jnp.int32)

      for r in range(embed // 8):
        for s in range(8):
          d = r * 8 + s
          bd = jnp.full((16,), d, jnp.int32)
          pd = plsc.load_gather(pos_v, [bl, bd])
          for q in range(8):
            v = plsc.load_gather(g_v, [iota + 16 * q, bd])
            t_v[r, s, pl.ds(16 * q, 16)] = v + pd
      pltpu.sync_copy(t_v, out_hbm.at[l, :, wid, :, :])
      return ()

    lax.fori_loop(0, seq_len, body, (), unroll=False)

  return k


def kernel(inputs, token_table, pos_table):
  batch, seq_len = inputs.shape
  vocab, embed = token_table.shape
  idx = inputs.transpose(1, 0).reshape(batch * seq_len).astype(jnp.int32)
  k = _make_kernel(batch, seq_len, vocab, embed)
  o5 = k(token_table, idx, pos_table)
  return o5.transpose(2, 4, 0, 1, 3).reshape(batch, seq_len, embed)


# speedup vs baseline: 1.2178x; 1.2178x over previous
import functools

import jax
import jax.numpy as jnp
from jax import lax
from jax.experimental import pallas as pl
from jax.experimental.pallas import tpu as pltpu
from jax.experimental.pallas import tpu_sc as plsc

NUM_CORES = 2
NUM_SUBCORES = 16
NUM_WORKERS = NUM_CORES * NUM_SUBCORES
GRP = 8


@functools.cache
def _make_kernel(batch, seq_len, vocab, embed):
  assert batch == 128 * NUM_WORKERS and embed == 32 and seq_len % GRP == 0
  n_grp = seq_len // GRP
  grp_rows = GRP * 128
  mesh = plsc.VectorSubcoreMesh(
      core_axis_name="c", subcore_axis_name="s",
      num_cores=NUM_CORES, num_subcores=NUM_SUBCORES)

  @functools.partial(
      pl.kernel,
      out_type=jax.ShapeDtypeStruct(
          (seq_len, embed // 8, batch // 128, 8, 128), jnp.float32),
      mesh=mesh,
      compiler_params=pltpu.CompilerParams(
          use_tc_tiling_on_sc=False, needs_layout_passes=False),
      scratch_types=[
          pltpu.VMEM((2 * grp_rows,), jnp.int32),
          pltpu.VMEM((2 * grp_rows, embed), jnp.float32),
          pltpu.VMEM((2, embed // 8, 8, 128), jnp.float32),
          pltpu.VMEM((seq_len, embed), jnp.float32),
          pltpu.SemaphoreType.DMA((2,)),
          pltpu.SemaphoreType.DMA((2,)),
          pltpu.SemaphoreType.DMA((2,)),
      ],
  )
  def k(table_hbm, idx_hbm, pos_hbm, out_hbm, idx_v, g_v, t_v, pos_v,
        isem, gsem, wsem):
    wid = lax.axis_index("s") * NUM_CORES + lax.axis_index("c")
    pltpu.sync_copy(pos_hbm, pos_v)
    iota = lax.iota(jnp.int32, 16)

    def fire_idx(g, buf):
      for j in range(GRP):
        src_off = pl.multiple_of((g * GRP + j) * batch + wid * 128, 128)
        dst_off = pl.multiple_of(buf * grp_rows + j * 128, 128)
        pltpu.async_copy(idx_hbm.at[pl.ds(src_off, 128)],
                         idx_v.at[pl.ds(dst_off, 128)], isem.at[buf])

    def drain_idx(buf):
      off = pl.multiple_of(buf * grp_rows, 128)
      pltpu.make_async_copy(idx_hbm.at[pl.ds(0, grp_rows)],
                            idx_v.at[pl.ds(off, grp_rows)],
                            isem.at[buf]).wait()

    def fire_gather(buf):
      off = pl.multiple_of(buf * grp_rows, 128)
      pltpu.async_copy(table_hbm.at[idx_v.at[pl.ds(off, grp_rows)]],
                       g_v.at[pl.ds(off, grp_rows)], gsem.at[buf])

    def drain_gather(buf):
      off = pl.multiple_of(buf * grp_rows, 128)
      pltpu.make_async_copy(table_hbm.at[pl.ds(0, grp_rows)],
                            g_v.at[pl.ds(off, grp_rows)],
                            gsem.at[buf]).wait()

    def drain_write(tb):
      pltpu.make_async_copy(t_v.at[tb], out_hbm.at[0, :, wid, :, :],
                            wsem.at[tb]).wait()

    fire_idx(0, 0)
    drain_idx(0)
    fire_gather(0)
    fire_idx(1, 1)

    def body(g, _):
      buf = lax.rem(g, 2)
      nxt = 1 - buf

      @pl.when(g + 1 < n_grp)
      def _():
        drain_idx(nxt)
        fire_gather(nxt)

      drain_gather(buf)

      @pl.when(g + 2 < n_grp)
      def _():
        fire_idx(g + 2, buf)

      row0 = buf * grp_rows
      for j in range(GRP):
        l = g * GRP + j
        tb = j % 2
        if j >= 2:
          drain_write(tb)
        else:
          @pl.when(g > 0)
          def _():
            drain_write(tb)
        bl = jnp.broadcast_to(l, (16,)).astype(jnp.int32)
        base = jnp.broadcast_to(row0 + j * 128, (16,)).astype(jnp.int32)

        def rbody(r, _):
          for s in range(8):
            bd = jnp.broadcast_to(r * 8 + s, (16,)).astype(jnp.int32)
            pd = plsc.load_gather(pos_v, [bl, bd])
            for q in range(8):
              v = plsc.load_gather(g_v, [base + (16 * q + iota), bd])
              t_v[tb, r, s, pl.ds(16 * q, 16)] = v + pd
          return ()

        lax.fori_loop(0, embed // 8, rbody, (), unroll=False)
        pltpu.async_copy(t_v.at[tb], out_hbm.at[l, :, wid, :, :],
                         wsem.at[tb])
      return ()

    lax.fori_loop(0, n_grp, body, (), unroll=False)
    drain_write(0)
    drain_write(1)

  return k


def kernel(inputs, token_table, pos_table):
  batch, seq_len = inputs.shape
  vocab, embed = token_table.shape
  idx = inputs.transpose(1, 0).reshape(batch * seq_len).astype(jnp.int32)
  k = _make_kernel(batch, seq_len, vocab, embed)
  o5 = k(token_table, idx, pos_table)
  return o5.transpose(2, 4, 0, 1, 3).reshape(batch, seq_len, embed)


# speedup vs baseline: 1.3659x; 1.1216x over previous
import functools

import jax
import jax.numpy as jnp
from jax import lax
from jax.experimental import pallas as pl
from jax.experimental.pallas import tpu as pltpu
from jax.experimental.pallas import tpu_sc as plsc

NUM_CORES = 2
NUM_SUBCORES = 16
NUM_WORKERS = NUM_CORES * NUM_SUBCORES
GRP = 8


@functools.cache
def _make_kernel(batch, seq_len, vocab, embed):
  assert batch == 128 * NUM_WORKERS and embed == 32 and seq_len % GRP == 0
  n_grp = seq_len // GRP
  grp_rows = GRP * 128
  mesh = plsc.VectorSubcoreMesh(
      core_axis_name="c", subcore_axis_name="s",
      num_cores=NUM_CORES, num_subcores=NUM_SUBCORES)

  @functools.partial(
      pl.kernel,
      out_type=jax.ShapeDtypeStruct(
          (seq_len, embed // 8, batch // 128, 8, 128), jnp.float32),
      mesh=mesh,
      compiler_params=pltpu.CompilerParams(
          use_tc_tiling_on_sc=False, needs_layout_passes=False),
      scratch_types=[
          pltpu.VMEM((2 * grp_rows,), jnp.int32),
          pltpu.VMEM((2 * grp_rows, embed), jnp.float32),
          pltpu.VMEM((2, embed // 8, 8, 128), jnp.float32),
          pltpu.VMEM((seq_len, embed), jnp.float32),
          pltpu.SemaphoreType.DMA((2,)),
          pltpu.SemaphoreType.DMA((2,)),
          pltpu.SemaphoreType.DMA((2,)),
      ],
  )
  def k(table_hbm, idx_hbm, pos_hbm, out_hbm, idx_v, g_v, t_v, pos_v,
        isem, gsem, wsem):
    wid = lax.axis_index("s") * NUM_CORES + lax.axis_index("c")
    pltpu.sync_copy(pos_hbm, pos_v)
    iota = lax.iota(jnp.int32, 16)

    def fire_idx(g, buf):
      for j in range(GRP):
        src_off = pl.multiple_of((g * GRP + j) * batch + wid * 128, 128)
        dst_off = pl.multiple_of(buf * grp_rows + j * 128, 128)
        pltpu.async_copy(idx_hbm.at[pl.ds(src_off, 128)],
                         idx_v.at[pl.ds(dst_off, 128)], isem.at[buf])

    def drain_idx(buf):
      off = pl.multiple_of(buf * grp_rows, 128)
      pltpu.make_async_copy(idx_hbm.at[pl.ds(0, grp_rows)],
                            idx_v.at[pl.ds(off, grp_rows)],
                            isem.at[buf]).wait()

    def fire_gather(buf):
      off = pl.multiple_of(buf * grp_rows, 128)
      pltpu.async_copy(table_hbm.at[idx_v.at[pl.ds(off, grp_rows)]],
                       g_v.at[pl.ds(off, grp_rows)], gsem.at[buf])

    def drain_gather(buf):
      off = pl.multiple_of(buf * grp_rows, 128)
      pltpu.make_async_copy(table_hbm.at[pl.ds(0, grp_rows)],
                            g_v.at[pl.ds(off, grp_rows)],
                            gsem.at[buf]).wait()

    def drain_write(tb):
      pltpu.make_async_copy(t_v.at[tb], out_hbm.at[0, :, wid, :, :],
                            wsem.at[tb]).wait()

    fire_idx(0, 0)
    drain_idx(0)
    fire_gather(0)
    fire_idx(1, 1)

    def body(g, _):
      buf = lax.rem(g, 2)
      nxt = 1 - buf

      @pl.when(g + 1 < n_grp)
      def _():
        drain_idx(nxt)
        fire_gather(nxt)

      drain_gather(buf)

      @pl.when(g + 2 < n_grp)
      def _():
        fire_idx(g + 2, buf)

      row0 = buf * grp_rows
      for j in range(GRP):
        l = g * GRP + j
        tb = j % 2
        if j >= 2:
          drain_write(tb)
        else:
          @pl.when(g > 0)
          def _():
            drain_write(tb)
        bl = jnp.broadcast_to(l, (16,)).astype(jnp.int32)
        base = jnp.broadcast_to(row0 + j * 128, (16,)).astype(jnp.int32)
        rows_q = [base + (16 * q + iota) for q in range(8)]

        @plsc.parallel_loop(0, embed // 8)
        def _(r):
          for s in range(8):
            bd = jnp.broadcast_to(r * 8 + s, (16,)).astype(jnp.int32)
            pd = plsc.load_gather(pos_v, [bl, bd])
            for q in range(8):
              v = plsc.load_gather(g_v, [rows_q[q], bd])
              t_v[tb, r, s, pl.ds(16 * q, 16)] = v + pd
        pltpu.async_copy(t_v.at[tb], out_hbm.at[l, :, wid, :, :],
                         wsem.at[tb])
      return ()

    lax.fori_loop(0, n_grp, body, (), unroll=False)
    drain_write(0)
    drain_write(1)

  return k


def kernel(inputs, token_table, pos_table):
  batch, seq_len = inputs.shape
  vocab, embed = token_table.shape
  idx = inputs.transpose(1, 0).reshape(batch * seq_len).astype(jnp.int32)
  k = _make_kernel(batch, seq_len, vocab, embed)
  o5 = k(token_table, idx, pos_table)
  return o5.transpose(2, 4, 0, 1, 3).reshape(batch, seq_len, embed)
